# baseline (device time: 76465 ns/iter reference)
import jax
import jax.numpy as jnp
from jax import lax
from jax.experimental import pallas as pl
from jax.experimental.pallas import tpu as pltpu

N_DEV = 4
SCALE = 0.08838834764831843
BLK = 64
PHASES = 4
N_WAVES = 4
N_STEPS = 2 * (N_DEV - 1)


def _wo_allreduce(ctx, Wo):
    S, D = ctx.shape
    Dl = Wo.shape[0] // N_DEV
    C = S // N_DEV
    H = C // 2
    Hw = H // N_WAVES

    def body(ctx_ref, wo_hbm, out_ref, wo_f32, wo_bf, comm_ref,
             copy_sems, cw_send, cw_recv, ccw_send, ccw_recv):
        my = lax.axis_index("i")
        left = (my - 1) % N_DEV
        right = (my + 1) % N_DEV

        wo_dma = pltpu.make_async_copy(
            wo_hbm.at[pl.ds(my * Dl, Dl), :], wo_f32,
            copy_sems.at[8 * N_WAVES]
        )
        wo_dma.start()

        barrier_sem = pltpu.get_barrier_semaphore()
        for nbr in [left, right]:
            pl.semaphore_signal(
                barrier_sem, inc=1,
                device_id=(nbr,), device_id_type=pl.DeviceIdType.MESH,
            )
        pl.semaphore_wait(barrier_sem, 2)

        wo_dma.wait()
        wo_bf[...] = wo_f32[...].astype(jnp.bfloat16)

        def pblock(row_start, n):
            a = ctx_ref[pl.ds(row_start, n), :]
            return jnp.dot(
                a, wo_bf[...], preferred_element_type=jnp.float32
            ).astype(jnp.bfloat16)

        def launch(k, wv):
            t0 = wv * Hw
            b0 = H + wv * Hw
            sem = wv * N_STEPS + k
            cw = pltpu.make_async_remote_copy(
                src_ref=comm_ref.at[k, t0:t0 + Hw, :],
                dst_ref=comm_ref.at[k + 1, t0:t0 + Hw, :],
                send_sem=cw_send.at[sem],
                recv_sem=cw_recv.at[sem],
                device_id=(right,),
                device_id_type=pl.DeviceIdType.MESH,
            )
            ccw = pltpu.make_async_remote_copy(
                src_ref=comm_ref.at[k, b0:b0 + Hw, :],
                dst_ref=comm_ref.at[k + 1, b0:b0 + Hw, :],
                send_sem=ccw_send.at[sem],
                recv_sem=ccw_recv.at[sem],
                device_id=(left,),
                device_id_type=pl.DeviceIdType.MESH,
            )
            cw.start()
            ccw.start()
            return cw, ccw

        infl = {}
        for wv in range(N_WAVES):
            t0 = wv * Hw
            b0 = H + wv * Hw
            comm_ref[0, t0:t0 + Hw, :] = pblock(my * C + t0, Hw)
            comm_ref[0, b0:b0 + Hw, :] = pblock(my * C + b0, Hw)
            infl[wv] = launch(0, wv)

        pending = []

        def out_copy(slot, v0, n, dst_row):
            cp = pltpu.make_async_copy(
                comm_ref.at[slot, v0:v0 + n, :],
                out_ref.at[pl.ds(dst_row, n), :],
                copy_sems.at[len(pending)],
            )
            cp.start()
            pending.append(cp)

        for k in range(N_STEPS):
            for wv in range(N_WAVES):
                t0 = wv * Hw
                b0 = H + wv * Hw
                cw, ccw = infl.pop(wv)
                if k < N_DEV - 1:
                    c_cw = (my - 1 - k) % N_DEV
                    c_ccw = (my + 1 + k) % N_DEV
                    vt = pblock(c_cw * C + t0, Hw)
                    vb = pblock(c_ccw * C + b0, Hw)
                    cw.wait()
                    ccw.wait()
                    comm_ref[k + 1, t0:t0 + Hw, :] = (
                        comm_ref[k + 1, t0:t0 + Hw, :] + vt
                    )
                    comm_ref[k + 1, b0:b0 + Hw, :] = (
                        comm_ref[k + 1, b0:b0 + Hw, :] + vb
                    )
                    infl[wv] = launch(k + 1, wv)
                else:
                    cw.wait()
                    ccw.wait()
                    if k < N_STEPS - 1:
                        infl[wv] = launch(k + 1, wv)
                    t = k - (N_DEV - 1)
                    if t == 0:
                        own_cw = (my + 1) % N_DEV
                        own_ccw = (my - 1) % N_DEV
                        out_copy(N_DEV - 1, t0, Hw, own_cw * C + t0)
                        out_copy(N_DEV - 1, b0, Hw, own_ccw * C + b0)
                    c1 = (my - t) % N_DEV
                    c2 = (my + t) % N_DEV
                    out_copy(k + 1, t0, Hw, c1 * C + t0)
                    out_copy(k + 1, b0, Hw, c2 * C + b0)

        for cp in pending:
            cp.wait()

    n_sems = N_WAVES * N_STEPS
    return pl.pallas_call(
        body,
        out_shape=jax.ShapeDtypeStruct((S, D), jnp.bfloat16),
        in_specs=[
            pl.BlockSpec(memory_space=pltpu.VMEM),
            pl.BlockSpec(memory_space=pl.ANY),
        ],
        out_specs=pl.BlockSpec(memory_space=pl.ANY),
        scratch_shapes=[
            pltpu.VMEM((Dl, D), jnp.float32),
            pltpu.VMEM((Dl, D), jnp.bfloat16),
            pltpu.VMEM((N_STEPS + 1, C, D), jnp.bfloat16),
            pltpu.SemaphoreType.DMA((8 * N_WAVES + 1,)),
            pltpu.SemaphoreType.DMA((n_sems,)),
            pltpu.SemaphoreType.DMA((n_sems,)),
            pltpu.SemaphoreType.DMA((n_sems,)),
            pltpu.SemaphoreType.DMA((n_sems,)),
        ],
        compiler_params=pltpu.CompilerParams(collective_id=0),
    )(ctx, Wo)


def _sparse_attention(x2, Wq, K, V):
    S, d_model = x2.shape
    _, Hl, Dh = K.shape
    Dl = Hl * Dh
    n_blk = S // BLK
    m = n_blk // PHASES

    def rows(p, j):
        return (j * PHASES + p) * BLK

    def body(x_ref, wq_hbm, k_ref, v_ref, out_ref, wq_f32, dma_sem):
        my = lax.axis_index("i")
        wq_dma = pltpu.make_async_copy(
            wq_hbm.at[:, pl.ds(my * Dl, Dl)], wq_f32, dma_sem
        )
        wq_dma.start()
        wq = None
        for p in range(PHASES):
            xp = jnp.concatenate(
                [x_ref[rows(p, j):rows(p, j) + BLK, :] for j in range(m)]
            ).astype(jnp.bfloat16)
            kp = jnp.concatenate(
                [k_ref[rows(p, j):rows(p, j) + BLK, :, :] for j in range(m)]
            ).astype(jnp.bfloat16)
            vp = jnp.concatenate(
                [v_ref[rows(p, j):rows(p, j) + BLK, :, :] for j in range(m)]
            ).astype(jnp.bfloat16)
            if wq is None:
                wq_dma.wait()
                wq = (wq_f32[...] * SCALE).astype(jnp.bfloat16)
            qp = jnp.dot(
                xp, wq, preferred_element_type=jnp.float32
            ).astype(jnp.bfloat16)
            ctx_h = []
            for h in range(Hl):
                qh = qp[:, h * Dh:(h + 1) * Dh]
                s = lax.dot_general(
                    qh, kp[:, h, :],
                    (((1,), (1,)), ((), ())),
                    preferred_element_type=jnp.float32,
                )
                e = jnp.exp(s)
                denom = jnp.sum(e, axis=-1, keepdims=True)
                ctx_un = jnp.dot(
                    e.astype(jnp.bfloat16), vp[:, h, :],
                    preferred_element_type=jnp.float32,
                )
                ctx_h.append((ctx_un / denom).astype(jnp.bfloat16))
            ctx_p = jnp.concatenate(ctx_h, axis=1)
            for j in range(m):
                out_ref[rows(p, j):rows(p, j) + BLK, :] = (
                    ctx_p[j * BLK:(j + 1) * BLK, :]
                )

    return pl.pallas_call(
        body,
        out_shape=jax.ShapeDtypeStruct((S, Dl), jnp.bfloat16),
        in_specs=[
            pl.BlockSpec(memory_space=pltpu.VMEM),
            pl.BlockSpec(memory_space=pl.ANY),
            pl.BlockSpec(memory_space=pltpu.VMEM),
            pl.BlockSpec(memory_space=pltpu.VMEM),
        ],
        out_specs=pl.BlockSpec(memory_space=pltpu.VMEM),
        scratch_shapes=[
            pltpu.VMEM((d_model, Dl), jnp.float32),
            pltpu.SemaphoreType.DMA,
        ],
    )(x2, Wq, K, V)


def kernel(x, Wq, K_ext, V_ext, Wo):
    ctx = _sparse_attention(x[0], Wq, K_ext[0], V_ext[0])
    out = _wo_allreduce(ctx, Wo)
    return out[None]


# device time: 75512 ns/iter; 1.0126x vs baseline; 1.0126x over previous
import jax
import jax.numpy as jnp
from jax import lax
from jax.experimental import pallas as pl
from jax.experimental.pallas import tpu as pltpu

N_DEV = 4
SCALE = 0.08838834764831843
BLK = 64
PHASES = 4
N_WAVES = 2
N_STEPS = 2 * (N_DEV - 1)


def _wo_allreduce(ctx, Wo):
    S, D = ctx.shape
    Dl = Wo.shape[0] // N_DEV
    C = S // N_DEV
    H = C // 2
    Hw = H // N_WAVES

    def body(ctx_ref, wo_hbm, out_ref, wo_f32, wo_bf, comm_ref,
             copy_sems, cw_send, cw_recv, ccw_send, ccw_recv):
        my = lax.axis_index("i")
        left = (my - 1) % N_DEV
        right = (my + 1) % N_DEV

        wo_dma = pltpu.make_async_copy(
            wo_hbm.at[pl.ds(my * Dl, Dl), :], wo_f32,
            copy_sems.at[8 * N_WAVES]
        )
        wo_dma.start()

        barrier_sem = pltpu.get_barrier_semaphore()
        for nbr in [left, right]:
            pl.semaphore_signal(
                barrier_sem, inc=1,
                device_id=(nbr,), device_id_type=pl.DeviceIdType.MESH,
            )
        pl.semaphore_wait(barrier_sem, 2)

        wo_dma.wait()
        wo_bf[...] = wo_f32[...].astype(jnp.bfloat16)

        def pblock(row_start, n):
            a = ctx_ref[pl.ds(row_start, n), :]
            return jnp.dot(
                a, wo_bf[...], preferred_element_type=jnp.float32
            ).astype(jnp.bfloat16)

        def launch(k, wv):
            t0 = wv * Hw
            b0 = H + wv * Hw
            sem = wv * N_STEPS + k
            cw = pltpu.make_async_remote_copy(
                src_ref=comm_ref.at[k, t0:t0 + Hw, :],
                dst_ref=comm_ref.at[k + 1, t0:t0 + Hw, :],
                send_sem=cw_send.at[sem],
                recv_sem=cw_recv.at[sem],
                device_id=(right,),
                device_id_type=pl.DeviceIdType.MESH,
            )
            ccw = pltpu.make_async_remote_copy(
                src_ref=comm_ref.at[k, b0:b0 + Hw, :],
                dst_ref=comm_ref.at[k + 1, b0:b0 + Hw, :],
                send_sem=ccw_send.at[sem],
                recv_sem=ccw_recv.at[sem],
                device_id=(left,),
                device_id_type=pl.DeviceIdType.MESH,
            )
            cw.start()
            ccw.start()
            return cw, ccw

        infl = {}
        for wv in range(N_WAVES):
            t0 = wv * Hw
            b0 = H + wv * Hw
            comm_ref[0, t0:t0 + Hw, :] = pblock(my * C + t0, Hw)
            comm_ref[0, b0:b0 + Hw, :] = pblock(my * C + b0, Hw)
            infl[wv] = launch(0, wv)

        pending = []

        def out_copy(slot, v0, n, dst_row):
            cp = pltpu.make_async_copy(
                comm_ref.at[slot, v0:v0 + n, :],
                out_ref.at[pl.ds(dst_row, n), :],
                copy_sems.at[len(pending)],
            )
            cp.start()
            pending.append(cp)

        for k in range(N_STEPS):
            for wv in range(N_WAVES):
                t0 = wv * Hw
                b0 = H + wv * Hw
                cw, ccw = infl.pop(wv)
                if k < N_DEV - 1:
                    c_cw = (my - 1 - k) % N_DEV
                    c_ccw = (my + 1 + k) % N_DEV
                    vt = pblock(c_cw * C + t0, Hw)
                    vb = pblock(c_ccw * C + b0, Hw)
                    cw.wait()
                    ccw.wait()
                    comm_ref[k + 1, t0:t0 + Hw, :] = (
                        comm_ref[k + 1, t0:t0 + Hw, :] + vt
                    )
                    comm_ref[k + 1, b0:b0 + Hw, :] = (
                        comm_ref[k + 1, b0:b0 + Hw, :] + vb
                    )
                    infl[wv] = launch(k + 1, wv)
                else:
                    cw.wait()
                    ccw.wait()
                    if k < N_STEPS - 1:
                        infl[wv] = launch(k + 1, wv)
                    t = k - (N_DEV - 1)
                    if t == 0:
                        own_cw = (my + 1) % N_DEV
                        own_ccw = (my - 1) % N_DEV
                        out_copy(N_DEV - 1, t0, Hw, own_cw * C + t0)
                        out_copy(N_DEV - 1, b0, Hw, own_ccw * C + b0)
                    c1 = (my - t) % N_DEV
                    c2 = (my + t) % N_DEV
                    out_copy(k + 1, t0, Hw, c1 * C + t0)
                    out_copy(k + 1, b0, Hw, c2 * C + b0)

        for cp in pending:
            cp.wait()

    n_sems = N_WAVES * N_STEPS
    return pl.pallas_call(
        body,
        out_shape=jax.ShapeDtypeStruct((S, D), jnp.bfloat16),
        in_specs=[
            pl.BlockSpec(memory_space=pltpu.VMEM),
            pl.BlockSpec(memory_space=pl.ANY),
        ],
        out_specs=pl.BlockSpec(memory_space=pl.ANY),
        scratch_shapes=[
            pltpu.VMEM((Dl, D), jnp.float32),
            pltpu.VMEM((Dl, D), jnp.bfloat16),
            pltpu.VMEM((N_STEPS + 1, C, D), jnp.bfloat16),
            pltpu.SemaphoreType.DMA((8 * N_WAVES + 1,)),
            pltpu.SemaphoreType.DMA((n_sems,)),
            pltpu.SemaphoreType.DMA((n_sems,)),
            pltpu.SemaphoreType.DMA((n_sems,)),
            pltpu.SemaphoreType.DMA((n_sems,)),
        ],
        compiler_params=pltpu.CompilerParams(collective_id=0),
    )(ctx, Wo)


def _sparse_attention(x2, Wq, K, V):
    S, d_model = x2.shape
    _, Hl, Dh = K.shape
    Dl = Hl * Dh
    n_blk = S // BLK
    m = n_blk // PHASES

    def rows(p, j):
        return (j * PHASES + p) * BLK

    def body(x_ref, wq_hbm, k_ref, v_ref, out_ref, wq_f32, dma_sem):
        my = lax.axis_index("i")
        wq_dma = pltpu.make_async_copy(
            wq_hbm.at[:, pl.ds(my * Dl, Dl)], wq_f32, dma_sem
        )
        wq_dma.start()
        wq = None
        for p in range(PHASES):
            xp = jnp.concatenate(
                [x_ref[rows(p, j):rows(p, j) + BLK, :] for j in range(m)]
            ).astype(jnp.bfloat16)
            kp = jnp.concatenate(
                [k_ref[rows(p, j):rows(p, j) + BLK, :, :] for j in range(m)]
            ).astype(jnp.bfloat16)
            vp = jnp.concatenate(
                [v_ref[rows(p, j):rows(p, j) + BLK, :, :] for j in range(m)]
            ).astype(jnp.bfloat16)
            if wq is None:
                wq_dma.wait()
                wq = (wq_f32[...] * SCALE).astype(jnp.bfloat16)
            qp = jnp.dot(
                xp, wq, preferred_element_type=jnp.float32
            ).astype(jnp.bfloat16)
            ctx_h = []
            for h in range(Hl):
                qh = qp[:, h * Dh:(h + 1) * Dh]
                s = lax.dot_general(
                    qh, kp[:, h, :],
                    (((1,), (1,)), ((), ())),
                    preferred_element_type=jnp.float32,
                )
                e = jnp.exp(s)
                denom = jnp.sum(e, axis=-1, keepdims=True)
                ctx_un = jnp.dot(
                    e.astype(jnp.bfloat16), vp[:, h, :],
                    preferred_element_type=jnp.float32,
                )
                ctx_h.append((ctx_un / denom).astype(jnp.bfloat16))
            ctx_p = jnp.concatenate(ctx_h, axis=1)
            for j in range(m):
                out_ref[rows(p, j):rows(p, j) + BLK, :] = (
                    ctx_p[j * BLK:(j + 1) * BLK, :]
                )

    return pl.pallas_call(
        body,
        out_shape=jax.ShapeDtypeStruct((S, Dl), jnp.bfloat16),
        in_specs=[
            pl.BlockSpec(memory_space=pltpu.VMEM),
            pl.BlockSpec(memory_space=pl.ANY),
            pl.BlockSpec(memory_space=pltpu.VMEM),
            pl.BlockSpec(memory_space=pltpu.VMEM),
        ],
        out_specs=pl.BlockSpec(memory_space=pltpu.VMEM),
        scratch_shapes=[
            pltpu.VMEM((d_model, Dl), jnp.float32),
            pltpu.SemaphoreType.DMA,
        ],
    )(x2, Wq, K, V)


def kernel(x, Wq, K_ext, V_ext, Wo):
    ctx = _sparse_attention(x[0], Wq, K_ext[0], V_ext[0])
    out = _wo_allreduce(ctx, Wo)
    return out[None]
